# Initial kernel scaffold; baseline (speedup 1.0000x reference)
import jax
import jax.numpy as jnp
from jax.experimental import pallas as pl


def kernel(user_index, theta_user_item, alpha_item, theta_user_cat, alpha_category, lambda_weight):
    raise NotImplementedError("write your pallas kernel here")


# speedup vs baseline: 3.9440x; 3.9440x over previous
import numpy as np
import jax
import jax.numpy as jnp
from jax import lax
from jax.experimental import pallas as pl
from jax.experimental.pallas import tpu as pltpu
from jax.experimental.pallas import tpu_sc as plsc

_S = 8192
_I = 1000
_C = 50
_G = 20
_D = 128
_BS = 512
_NW = 32
_BW = _S // _NW

_SEG = np.arange(_I) // _G
_MSUM_NP = (_SEG[:, None] == np.arange(_C)[None, :]).astype(np.float32)
_MEXP_NP = _MSUM_NP.T.copy()


def _sc_gather_body(t1_hbm, t2_hbm, idx_hbm, o1_hbm, o2_hbm, idx_v, rows_v, sem):
    wid = lax.axis_index("s") * 2 + lax.axis_index("c")
    base = wid * _BW
    pltpu.sync_copy(idx_hbm.at[pl.ds(base, _BW)], idx_v)
    pltpu.async_copy(t1_hbm.at[idx_v], rows_v, sem).wait()
    pltpu.sync_copy(rows_v, o1_hbm.at[pl.ds(base, _BW)])
    pltpu.async_copy(t2_hbm.at[idx_v], rows_v, sem).wait()
    pltpu.sync_copy(rows_v, o2_hbm.at[pl.ds(base, _BW)])


def _sc_gather(t1, t2, idx):
    mesh = plsc.VectorSubcoreMesh(core_axis_name="c", subcore_axis_name="s")
    k = pl.kernel(
        _sc_gather_body,
        out_type=[
            jax.ShapeDtypeStruct((_S, _D), jnp.float32),
            jax.ShapeDtypeStruct((_S, _D), jnp.float32),
        ],
        mesh=mesh,
        scratch_types=[
            pltpu.VMEM((_BW,), jnp.int32),
            pltpu.VMEM((_BW, _D), jnp.float32),
            pltpu.SemaphoreType.DMA,
        ],
    )
    return k(t1, t2, idx)


def _hi_lo(x):
    hi = x.astype(jnp.bfloat16)
    lo = (x - hi.astype(jnp.float32)).astype(jnp.bfloat16)
    return hi, lo


def _mask_dot(x, m_ref):
    hi, lo = _hi_lo(x)
    m = m_ref[...]
    return (jnp.dot(hi, m, preferred_element_type=jnp.float32)
            + jnp.dot(lo, m, preferred_element_type=jnp.float32))


def _tc_body(tu_ref, tc_ref, aT_ref, acT_ref, invl_ref, lam_ref,
             msum_ref, mexp_ref, tus_ref, tcs_ref, out_ref, lsec_ref):
    f32 = jnp.float32

    @pl.when(pl.program_id(0) == 0)
    def _prologue():
        tus = tus_ref[...].astype(jnp.bfloat16)
        ys = jnp.dot(tus, aT_ref[...], preferred_element_type=f32) * invl_ref[...]
        ssum = _mask_dot(jnp.exp(ys), msum_ref)
        inc = jnp.log(ssum)
        w = jnp.dot(tcs_ref[...].astype(jnp.bfloat16), acT_ref[...],
                    preferred_element_type=f32)
        logit = w + lam_ref[...] * inc
        lsec_ref[...] = jnp.log(jnp.sum(jnp.exp(logit), axis=0, keepdims=True))

    tu = tu_ref[...].astype(jnp.bfloat16)
    ys = jnp.dot(tu, aT_ref[...], preferred_element_type=f32) * invl_ref[...]
    ssum = _mask_dot(jnp.exp(ys), msum_ref)
    inc = jnp.log(ssum)
    w = jnp.dot(tc_ref[...].astype(jnp.bfloat16), acT_ref[...],
                preferred_element_type=f32)
    a = (lam_ref[...] - 1.0) * inc + w - lsec_ref[...]
    out_ref[...] = ys + _mask_dot(a, mexp_ref)


def _tc_grid_args():
    full = lambda b: (0, 0)
    in_specs = [
        pl.BlockSpec((_BS, _D), lambda b: (b, 0)),
        pl.BlockSpec((_BS, _D), lambda b: (b, 0)),
        pl.BlockSpec((_D, _I), full),
        pl.BlockSpec((_D, _C), full),
        pl.BlockSpec((1, _I), full),
        pl.BlockSpec((1, _C), full),
        pl.BlockSpec((_I, _C), full),
        pl.BlockSpec((_C, _I), full),
        pl.BlockSpec((_C, _D), full),
        pl.BlockSpec((_C, _D), full),
    ]
    return dict(
        grid=(_S // _BS,),
        in_specs=in_specs,
        out_specs=pl.BlockSpec((_BS, _I), lambda b: (b, 0)),
        out_shape=jax.ShapeDtypeStruct((_S, _I), jnp.float32),
        scratch_shapes=[pltpu.VMEM((1, _C), jnp.float32)],
    )


def kernel(user_index, theta_user_item, alpha_item, theta_user_cat,
           alpha_category, lambda_weight):
    idx = user_index.astype(jnp.int32)
    tu_g, tc_g = _sc_gather(theta_user_item, theta_user_cat, idx)
    lam = lambda_weight.reshape(1, _C).astype(jnp.float32)
    invl = jnp.repeat(1.0 / lambda_weight, _G).reshape(1, _I).astype(jnp.float32)
    aT = alpha_item.T.astype(jnp.bfloat16)
    acT = alpha_category.T.astype(jnp.bfloat16)
    msum = jnp.asarray(_MSUM_NP).astype(jnp.bfloat16)
    mexp = jnp.asarray(_MEXP_NP).astype(jnp.bfloat16)
    tus = tu_g[0:_C * _G:_G]
    tcs = tc_g[0:_C * _G:_G]
    return pl.pallas_call(_tc_body, **_tc_grid_args())(
        tu_g, tc_g, aT, acT, invl, lam, msum, mexp, tus, tcs)


# speedup vs baseline: 4.5516x; 1.1540x over previous
import numpy as np
import jax
import jax.numpy as jnp
from jax import lax
from jax.experimental import pallas as pl
from jax.experimental.pallas import tpu as pltpu
from jax.experimental.pallas import tpu_sc as plsc

_S = 8192
_I = 1000
_C = 50
_G = 20
_D = 128
_BS = 512
_NW = 32
_BW = _S // _NW

_SEG = np.arange(_I) // _G
_MSUM_NP = (_SEG[:, None] == np.arange(_C)[None, :]).astype(np.float32)
_MEXP_NP = _MSUM_NP.T.copy()


def _sc_gather_body(t1_hbm, t2_hbm, idx_hbm, o1_hbm, o2_hbm, idx_v, rows_v, sem):
    wid = lax.axis_index("s") * 2 + lax.axis_index("c")
    base = wid * _BW
    pltpu.sync_copy(idx_hbm.at[pl.ds(base, _BW)], idx_v)
    pltpu.async_copy(t1_hbm.at[idx_v], rows_v, sem).wait()
    pltpu.sync_copy(rows_v, o1_hbm.at[pl.ds(base, _BW)])
    pltpu.async_copy(t2_hbm.at[idx_v], rows_v, sem).wait()
    pltpu.sync_copy(rows_v, o2_hbm.at[pl.ds(base, _BW)])


def _sc_gather(t1, t2, idx):
    mesh = plsc.VectorSubcoreMesh(core_axis_name="c", subcore_axis_name="s")
    k = pl.kernel(
        _sc_gather_body,
        out_type=[
            jax.ShapeDtypeStruct((_S, _D), jnp.float32),
            jax.ShapeDtypeStruct((_S, _D), jnp.float32),
        ],
        mesh=mesh,
        scratch_types=[
            pltpu.VMEM((_BW,), jnp.int32),
            pltpu.VMEM((_BW, _D), jnp.float32),
            pltpu.SemaphoreType.DMA,
        ],
    )
    return k(t1, t2, idx)


def _tc_body(tu_ref, tc_ref, aT_ref, acT_ref, invl_ref, lam_ref,
             msum_ref, mexp_ref, tus_ref, tcs_ref, out_ref, aTs_ref, lsec_ref):
    f32 = jnp.float32
    bf16 = jnp.bfloat16

    @pl.when(pl.program_id(0) == 0)
    def _prologue():
        aTs_ref[...] = (aT_ref[...] * invl_ref[...]).astype(bf16)
        tus = tus_ref[...].astype(bf16)
        ys = jnp.dot(tus, aTs_ref[...], preferred_element_type=f32)
        ssum = jnp.dot(jnp.exp(ys).astype(bf16), msum_ref[...],
                       preferred_element_type=f32)
        inc = jnp.log(ssum)
        w = jnp.dot(tcs_ref[...].astype(bf16), acT_ref[...],
                    preferred_element_type=f32)
        logit = w + lam_ref[...] * inc
        lsec_ref[...] = jnp.log(jnp.sum(jnp.exp(logit), axis=0, keepdims=True))

    tu = tu_ref[...].astype(bf16)
    ys = jnp.dot(tu, aTs_ref[...], preferred_element_type=f32)
    ssum = jnp.dot(jnp.exp(ys).astype(bf16), msum_ref[...],
                   preferred_element_type=f32)
    inc = jnp.log(ssum)
    w = jnp.dot(tc_ref[...].astype(bf16), acT_ref[...],
                preferred_element_type=f32)
    a = (lam_ref[...] - 1.0) * inc + w - lsec_ref[...]
    mu = jnp.mean(a, axis=1, keepdims=True)
    aexp = jnp.dot((a - mu).astype(bf16), mexp_ref[...],
                   preferred_element_type=f32)
    out_ref[...] = (ys + mu) + aexp


def _tc_grid_args():
    full = lambda b: (0, 0)
    in_specs = [
        pl.BlockSpec((_BS, _D), lambda b: (b, 0)),
        pl.BlockSpec((_BS, _D), lambda b: (b, 0)),
        pl.BlockSpec((_D, _I), full),
        pl.BlockSpec((_D, _C), full),
        pl.BlockSpec((1, _I), full),
        pl.BlockSpec((1, _C), full),
        pl.BlockSpec((_I, _C), full),
        pl.BlockSpec((_C, _I), full),
        pl.BlockSpec((_C, _D), full),
        pl.BlockSpec((_C, _D), full),
    ]
    return dict(
        grid=(_S // _BS,),
        in_specs=in_specs,
        out_specs=pl.BlockSpec((_BS, _I), lambda b: (b, 0)),
        out_shape=jax.ShapeDtypeStruct((_S, _I), jnp.float32),
        scratch_shapes=[pltpu.VMEM((_D, _I), jnp.bfloat16),
                        pltpu.VMEM((1, _C), jnp.float32)],
    )


def kernel(user_index, theta_user_item, alpha_item, theta_user_cat,
           alpha_category, lambda_weight):
    idx = user_index.astype(jnp.int32)
    tu_g, tc_g = _sc_gather(theta_user_item, theta_user_cat, idx)
    lam = lambda_weight.reshape(1, _C).astype(jnp.float32)
    invl = jnp.repeat(1.0 / lambda_weight, _G).reshape(1, _I).astype(jnp.float32)
    aT = alpha_item.T.astype(jnp.float32)
    acT = alpha_category.T.astype(jnp.bfloat16)
    msum = jnp.asarray(_MSUM_NP).astype(jnp.bfloat16)
    mexp = jnp.asarray(_MEXP_NP).astype(jnp.bfloat16)
    tus = tu_g[0:_C * _G:_G]
    tcs = tc_g[0:_C * _G:_G]
    return pl.pallas_call(_tc_body, **_tc_grid_args())(
        tu_g, tc_g, aT, acT, invl, lam, msum, mexp, tus, tcs)


# speedup vs baseline: 4.7477x; 1.0431x over previous
import numpy as np
import jax
import jax.numpy as jnp
from jax import lax
from jax.experimental import pallas as pl
from jax.experimental.pallas import tpu as pltpu
from jax.experimental.pallas import tpu_sc as plsc

_S = 8192
_I = 1000
_C = 50
_G = 20
_D = 128
_BS = 1024
_NW = 32
_BW = _S // _NW

_SEG = np.arange(_I) // _G
_MSUM_NP = (_SEG[:, None] == np.arange(_C)[None, :]).astype(np.float32)
_MEXP_NP = _MSUM_NP.T.copy()


def _sc_gather_body(t1_hbm, t2_hbm, idx_hbm, o1_hbm, o2_hbm, idx_v, rows_v, sem):
    wid = lax.axis_index("s") * 2 + lax.axis_index("c")
    base = wid * _BW
    pltpu.sync_copy(idx_hbm.at[pl.ds(base, _BW)], idx_v)
    pltpu.async_copy(t1_hbm.at[idx_v], rows_v, sem).wait()
    pltpu.sync_copy(rows_v, o1_hbm.at[pl.ds(base, _BW)])
    pltpu.async_copy(t2_hbm.at[idx_v], rows_v, sem).wait()
    pltpu.sync_copy(rows_v, o2_hbm.at[pl.ds(base, _BW)])


def _sc_gather(t1, t2, idx):
    mesh = plsc.VectorSubcoreMesh(core_axis_name="c", subcore_axis_name="s")
    k = pl.kernel(
        _sc_gather_body,
        out_type=[
            jax.ShapeDtypeStruct((_S, _D), jnp.float32),
            jax.ShapeDtypeStruct((_S, _D), jnp.float32),
        ],
        mesh=mesh,
        scratch_types=[
            pltpu.VMEM((_BW,), jnp.int32),
            pltpu.VMEM((_BW, _D), jnp.float32),
            pltpu.SemaphoreType.DMA,
        ],
    )
    return k(t1, t2, idx)


def _tc_body(tu_ref, tc_ref, aT_ref, acT_ref, invl_ref, lam_ref,
             msum_ref, mexp_ref, tus_ref, tcs_ref, out_ref, aTs_ref, lsec_ref):
    f32 = jnp.float32
    bf16 = jnp.bfloat16

    @pl.when(pl.program_id(0) == 0)
    def _prologue():
        aTs_ref[...] = (aT_ref[...] * invl_ref[...]).astype(bf16)
        tus = tus_ref[...].astype(bf16)
        ys = jnp.dot(tus, aTs_ref[...], preferred_element_type=f32)
        ssum = jnp.dot(jnp.exp(ys).astype(bf16), msum_ref[...],
                       preferred_element_type=f32)
        inc = jnp.log(ssum)
        w = jnp.dot(tcs_ref[...].astype(bf16), acT_ref[...],
                    preferred_element_type=f32)
        logit = w + lam_ref[...] * inc
        lsec_ref[...] = jnp.log(jnp.sum(jnp.exp(logit), axis=0, keepdims=True))

    tu = tu_ref[...].astype(bf16)
    ys = jnp.dot(tu, aTs_ref[...], preferred_element_type=f32)
    ssum = jnp.dot(jnp.exp(ys).astype(bf16), msum_ref[...],
                   preferred_element_type=f32)
    inc = jnp.log(ssum)
    w = jnp.dot(tc_ref[...].astype(bf16), acT_ref[...],
                preferred_element_type=f32)
    a = (lam_ref[...] - 1.0) * inc + w - lsec_ref[...]
    mu = jnp.mean(a, axis=1, keepdims=True)
    aexp = jnp.dot((a - mu).astype(bf16), mexp_ref[...],
                   preferred_element_type=f32)
    out_ref[...] = (ys + mu) + aexp


def _tc_grid_args():
    full = lambda b: (0, 0)
    in_specs = [
        pl.BlockSpec((_BS, _D), lambda b: (b, 0)),
        pl.BlockSpec((_BS, _D), lambda b: (b, 0)),
        pl.BlockSpec((_D, _I), full),
        pl.BlockSpec((_D, _C), full),
        pl.BlockSpec((1, _I), full),
        pl.BlockSpec((1, _C), full),
        pl.BlockSpec((_I, _C), full),
        pl.BlockSpec((_C, _I), full),
        pl.BlockSpec((_C, _D), full),
        pl.BlockSpec((_C, _D), full),
    ]
    return dict(
        grid=(_S // _BS,),
        in_specs=in_specs,
        out_specs=pl.BlockSpec((_BS, _I), lambda b: (b, 0)),
        out_shape=jax.ShapeDtypeStruct((_S, _I), jnp.float32),
        scratch_shapes=[pltpu.VMEM((_D, _I), jnp.bfloat16),
                        pltpu.VMEM((1, _C), jnp.float32)],
    )


def kernel(user_index, theta_user_item, alpha_item, theta_user_cat,
           alpha_category, lambda_weight):
    idx = user_index.astype(jnp.int32)
    tu_g, tc_g = _sc_gather(theta_user_item, theta_user_cat, idx)
    lam = lambda_weight.reshape(1, _C).astype(jnp.float32)
    invl = jnp.repeat(1.0 / lambda_weight, _G).reshape(1, _I).astype(jnp.float32)
    aT = alpha_item.T.astype(jnp.float32)
    acT = alpha_category.T.astype(jnp.bfloat16)
    msum = jnp.asarray(_MSUM_NP).astype(jnp.bfloat16)
    mexp = jnp.asarray(_MEXP_NP).astype(jnp.bfloat16)
    tus = tu_g[0:_C * _G:_G]
    tcs = tc_g[0:_C * _G:_G]
    return pl.pallas_call(_tc_body, **_tc_grid_args())(
        tu_g, tc_g, aT, acT, invl, lam, msum, mexp, tus, tcs)
